# Initial kernel scaffold; baseline (speedup 1.0000x reference)
import jax
import jax.numpy as jnp
from jax.experimental import pallas as pl


def kernel(input_ids, hidden_states, C0, C1, C2, C3, W, b):
    raise NotImplementedError("write your pallas kernel here")


# speedup vs baseline: 2.1845x; 2.1845x over previous
import functools
import jax
import jax.numpy as jnp
from jax import lax
from jax.experimental import pallas as pl
from jax.experimental.pallas import tpu as pltpu
from jax.experimental.pallas import tpu_sc as plsc

B = 1024
L = 200
D = 64
V = 100000
HOPS = 3
NT = 4

NC = 2
NS = 16
NW = NC * NS
N_IDS = B * L
IDS_PER_W = N_IDS // NW
CHUNK = 128
N_CHUNKS = IDS_PER_W // CHUNK


def _sc_gather_body(ids_hbm, t0, t1, t2, t3, o0, o1, o2, o3, idx_v, rows_v, sem):
    wid = lax.axis_index("s") * NC + lax.axis_index("c")
    base = wid * IDS_PER_W
    tables = [t0, t1, t2, t3]
    outs = [o0, o1, o2, o3]
    pltpu.sync_copy(ids_hbm.at[wid], idx_v)

    def chunk_step(i, carry):
        copies = []
        for t in range(NT):
            copies.append(
                pltpu.async_copy(tables[t].at[idx_v.at[i]], rows_v.at[t], sem)
            )
        for c in copies:
            c.wait()
        for t in range(NT):
            pltpu.sync_copy(rows_v.at[t], outs[t].at[pl.ds(base + i * CHUNK, CHUNK)])
        return carry

    lax.fori_loop(0, N_CHUNKS, chunk_step, 0)


@jax.jit
def _sc_gather(ids3, t0, t1, t2, t3):
    mesh = plsc.VectorSubcoreMesh(core_axis_name="c", subcore_axis_name="s")
    out_t = tuple(
        jax.ShapeDtypeStruct((N_IDS, D), jnp.float32) for _ in range(NT)
    )
    return pl.kernel(
        _sc_gather_body,
        out_type=out_t,
        mesh=mesh,
        scratch_types=[
            pltpu.VMEM((N_CHUNKS, CHUNK), jnp.int32),
            pltpu.VMEM((NT, CHUNK, D), jnp.float32),
            pltpu.SemaphoreType.DMA,
        ],
        compiler_params=pltpu.CompilerParams(use_tc_tiling_on_sc=False),
    )(ids3, t0, t1, t2, t3)


BB = 32


def _hops_body(h_ref, g0, g1, g2, g3, u_ref):
    u = h_ref[...]
    gs = [g0, g1, g2, g3]
    for hop in range(HOPS):
        ga = gs[hop][...]
        logits = jnp.sum(ga * u[:, None, :], axis=2)
        m = jnp.max(logits, axis=1, keepdims=True)
        e = jnp.exp(logits - m)
        p = e / jnp.sum(e, axis=1, keepdims=True)
        gc = gs[hop + 1][...]
        u = u + jnp.sum(gc * p[:, :, None], axis=1)
    u_ref[...] = u


@jax.jit
def _hops(hidden, g0, g1, g2, g3):
    gspec = pl.BlockSpec((BB, L, D), lambda i: (i, 0, 0))
    return pl.pallas_call(
        _hops_body,
        grid=(B // BB,),
        in_specs=[pl.BlockSpec((BB, D), lambda i: (i, 0))] + [gspec] * NT,
        out_specs=pl.BlockSpec((BB, D), lambda i: (i, 0)),
        out_shape=jax.ShapeDtypeStruct((B, D), jnp.float32),
    )(hidden, g0, g1, g2, g3)


VB = 2048


def _classifier_body(u_ref, w_ref, b_ref, o_ref):
    acc = lax.dot_general(
        u_ref[...], w_ref[...],
        dimension_numbers=(((1,), (1,)), ((), ())),
        preferred_element_type=jnp.float32,
    )
    o_ref[...] = jax.nn.sigmoid(acc + b_ref[...])


@jax.jit
def _classifier(u, W, b2):
    nvb = pl.cdiv(V, VB)
    return pl.pallas_call(
        _classifier_body,
        grid=(nvb,),
        in_specs=[
            pl.BlockSpec((B, D), lambda j: (0, 0)),
            pl.BlockSpec((VB, D), lambda j: (j, 0)),
            pl.BlockSpec((1, VB), lambda j: (0, j)),
        ],
        out_specs=pl.BlockSpec((B, VB), lambda j: (0, j)),
        out_shape=jax.ShapeDtypeStruct((B, V), jnp.float32),
    )(u, W, b2)


def kernel(input_ids, hidden_states, C0, C1, C2, C3, W, b):
    ids3 = input_ids.astype(jnp.int32).reshape(NW, N_CHUNKS, CHUNK)
    g0, g1, g2, g3 = _sc_gather(ids3, C0, C1, C2, C3)
    g0 = g0.reshape(B, L, D)
    g1 = g1.reshape(B, L, D)
    g2 = g2.reshape(B, L, D)
    g3 = g3.reshape(B, L, D)
    u = _hops(hidden_states, g0, g1, g2, g3)
    return _classifier(u, W, b.reshape(1, V))
